# Initial kernel scaffold; baseline (speedup 1.0000x reference)
import jax
import jax.numpy as jnp
from jax.experimental import pallas as pl


def kernel(feat, pos_emb_weight):
    raise NotImplementedError("write your pallas kernel here")


# speedup vs baseline: 1.6657x; 1.6657x over previous
import jax
import jax.numpy as jnp
from jax.experimental import pallas as pl

SEQ_BLOCK = 256


def _add_kernel(feat_ref, pos_ref, out_ref):
    out_ref[...] = feat_ref[...] + pos_ref[...][None, :, :]


def kernel(feat, pos_emb_weight):
    B, S, D = feat.shape
    pos = pos_emb_weight[:S]
    grid = (S // SEQ_BLOCK, B)
    return pl.pallas_call(
        _add_kernel,
        grid=grid,
        in_specs=[
            pl.BlockSpec((1, SEQ_BLOCK, D), lambda s, b: (b, s, 0)),
            pl.BlockSpec((SEQ_BLOCK, D), lambda s, b: (s, 0)),
        ],
        out_specs=pl.BlockSpec((1, SEQ_BLOCK, D), lambda s, b: (b, s, 0)),
        out_shape=jax.ShapeDtypeStruct((B, S, D), feat.dtype),
    )(feat, pos)


# speedup vs baseline: 1.7333x; 1.0406x over previous
import jax
import jax.numpy as jnp
from jax.experimental import pallas as pl

SEQ_BLOCK = 512


def _add_kernel(feat_ref, pos_ref, out_ref):
    out_ref[...] = feat_ref[...] + pos_ref[...][None, :, :]


def kernel(feat, pos_emb_weight):
    B, S, D = feat.shape
    pos = pos_emb_weight[:S]
    grid = (S // SEQ_BLOCK, B)
    return pl.pallas_call(
        _add_kernel,
        grid=grid,
        in_specs=[
            pl.BlockSpec((1, SEQ_BLOCK, D), lambda s, b: (b, s, 0)),
            pl.BlockSpec((SEQ_BLOCK, D), lambda s, b: (s, 0)),
        ],
        out_specs=pl.BlockSpec((1, SEQ_BLOCK, D), lambda s, b: (b, s, 0)),
        out_shape=jax.ShapeDtypeStruct((B, S, D), feat.dtype),
    )(feat, pos)
